# baseline (speedup 1.0000x reference)
import functools

import jax
import jax.numpy as jnp
from jax.experimental import pallas as pl
from jax.experimental.pallas import tpu as pltpu

NUM_ITEMS = 200
PAD = 0
NEG = -1000000000.0


def _spop_block(item_ref, *refs):
    ban_refs = refs[:-1]
    out_ref = refs[-1]
    B, S, _ = item_ref.shape
    C = NUM_ITEMS

    iota_c2 = jax.lax.broadcasted_iota(jnp.int32, (B, 1, C), 2)

    counts = jnp.zeros((B, 1, C), jnp.float32)
    last = jnp.full((B, 1, 1), -1, jnp.int32)
    for s in range(S):
        col = item_ref[:, s : s + 1, :]
        valid = col != PAD
        counts = counts + jnp.where((iota_c2 == col) & valid, 1.0, 0.0)
        last = jnp.where(valid, col, last)
    counts = counts - jnp.where(iota_c2 == last, 1.0, 0.0)

    m = jnp.max(counts, axis=2, keepdims=True)
    exprow = jnp.exp(counts - m)

    iota_cf = jax.lax.broadcasted_iota(jnp.int32, (B, S, C), 2).astype(
        jnp.float32
    )
    d = ban_refs[0][...] - iota_cf
    for bref in ban_refs[1:]:
        d = d * (bref[...] - iota_cf)
    banned = d == 0.0

    sum_unb = jnp.sum(
        jnp.where(banned, 0.0, exprow), axis=2, keepdims=True
    )
    lse = m + jnp.log(sum_unb)
    out_ref[...] = jnp.where(banned, NEG, counts) - lse


@functools.partial(jax.jit, static_argnames=("interpret",))
def _spop(ban_ids, item_ids, interpret=False):
    N, S, K = ban_ids.shape
    B = 128
    grid = (N // B,)
    item3 = item_ids[:, :, None]
    ban_cols = [
        ban_ids[:, :, k : k + 1].astype(jnp.float32) for k in range(K)
    ]
    spec_s1 = pl.BlockSpec((B, S, 1), lambda i: (i, 0, 0))
    pi = pl.pallas_call(
        _spop_block,
        grid=grid,
        in_specs=[spec_s1] * (1 + K),
        out_specs=pl.BlockSpec((B, S, NUM_ITEMS), lambda i: (i, 0, 0)),
        out_shape=jax.ShapeDtypeStruct((N, S, NUM_ITEMS), jnp.float32),
        compiler_params=pltpu.CompilerParams(
            dimension_semantics=("parallel",),
        ),
        interpret=interpret,
    )(item3, *ban_cols)
    return pi


def kernel(ban_ids, item_ids, aux1, aux2, aux3):
    pi = _spop(ban_ids, item_ids)
    n, s = item_ids.shape
    v = jnp.zeros((n, s, 1), jnp.float32)
    return (pi, v)


# speedup vs baseline: 1.3973x; 1.3973x over previous
import functools

import jax
import jax.numpy as jnp
from jax.experimental import pallas as pl
from jax.experimental.pallas import tpu as pltpu

NUM_ITEMS = 200
PAD = 0
NEG = -1000000000.0


def _spop_block(item_ref, ban_ref, out_ref):
    B, S = item_ref.shape
    K = ban_ref.shape[2]
    C = NUM_ITEMS

    iota_i = jax.lax.broadcasted_iota(jnp.int32, (B, C), 1)
    counts = jnp.zeros((B, C), jnp.float32)
    last = jnp.full((B, 1), -1, jnp.int32)
    for s in range(S):
        col = item_ref[:, s].reshape(B, 1)
        valid = col != PAD
        counts = counts + jnp.where((iota_i == col) & valid, 1.0, 0.0)
        last = jnp.where(valid, col, last)
    counts = counts - jnp.where(iota_i == last, 1.0, 0.0)

    m = jnp.max(counts, axis=1, keepdims=True)
    exp_row = jnp.exp(counts - m)

    iota_f = jax.lax.broadcasted_iota(jnp.int32, (B, S, C), 2).astype(
        jnp.float32
    )
    d = ban_ref[:, :, 0].astype(jnp.float32)[:, :, None] - iota_f
    for k in range(1, K):
        d = d * (ban_ref[:, :, k].astype(jnp.float32)[:, :, None] - iota_f)
    banned = d == 0.0

    sum_unb = jnp.sum(
        jnp.where(banned, 0.0, exp_row[:, None, :]), axis=2
    )
    lse = m + jnp.log(sum_unb)
    out_ref[...] = (
        jnp.where(banned, NEG, counts[:, None, :]) - lse[:, :, None]
    )


@functools.partial(jax.jit, static_argnames=("interpret",))
def _spop(ban_ids, item_ids, interpret=False):
    N, S = item_ids.shape
    K = ban_ids.shape[2]
    B = 128
    grid = (N // B,)
    pi = pl.pallas_call(
        _spop_block,
        grid=grid,
        in_specs=[
            pl.BlockSpec((B, S), lambda i: (i, 0)),
            pl.BlockSpec((B, S, K), lambda i: (i, 0, 0)),
        ],
        out_specs=pl.BlockSpec((B, S, NUM_ITEMS), lambda i: (i, 0, 0)),
        out_shape=jax.ShapeDtypeStruct((N, S, NUM_ITEMS), jnp.float32),
        compiler_params=pltpu.CompilerParams(
            dimension_semantics=("parallel",),
        ),
        interpret=interpret,
    )(item_ids, ban_ids)
    return pi


from jax import lax
from jax.experimental.pallas import tpu_sc as plsc

_NC, _NS, _L = 2, 16, 16
_NW = _NC * _NS
_SP = 32
_KP = 16
_SK = 20 * _KP
_CP = 208
_LN2 = 0.6931471805599453


def _log16(x):
    bits = lax.bitcast_convert_type(x, jnp.int32)
    e = lax.shift_right_logical(bits, 23) - 127
    f = lax.bitcast_convert_type(
        jnp.bitwise_or(jnp.bitwise_and(bits, 0x007FFFFF), 0x3F800000),
        jnp.float32,
    )
    big = f > 1.4142135
    f = jnp.where(big, f * 0.5, f)
    e = e + jnp.where(big, 1, 0)
    z = (f - 1.0) / (f + 1.0)
    z2 = z * z
    p = 2.0 * z * (1.0 + z2 * (1.0 / 3.0 + z2 * (0.2 + z2 * (1.0 / 7.0))))
    return e.astype(jnp.float32) * _LN2 + p


def _vred16(x, op):
    iota = lax.iota(jnp.int32, _L)
    for sh in (8, 4, 2, 1):
        x = op(x, x.at[iota ^ sh].get(mode="promise_in_bounds"))
    return x


def _sc_body(item_hbm, ban_hbm, out_hbm, items_v, bans_v, counts_v, out_v,
             cnt_v, sem0, sem1):
    N = item_hbm.shape[0] // _SP
    NB = N // _NW
    CH = 4
    CHW = CH * 20 * NUM_ITEMS
    wid = lax.axis_index("s") * _NC + lax.axis_index("c")
    base_n = wid * NB
    iota = lax.iota(jnp.int32, _L)

    pltpu.sync_copy(item_hbm.at[pl.ds(base_n * _SP, NB * _SP)], items_v)
    pltpu.sync_copy(ban_hbm.at[pl.ds(base_n * _SK, NB * _SK)], bans_v)
    for j in range(_CP // _L):
        cnt_v[pl.ds(j * _L, _L)] = jnp.zeros((_L,), jnp.float32)

    def body_a(n, c):
        cbase = n * _CP
        for j in range(_CP // _L):
            counts_v[pl.ds(cbase + j * _L, _L)] = jnp.where(
                iota + (j * _L) < NUM_ITEMS, 0.0, -100000.0
            )
        it0 = items_v[pl.ds(n * _SP, _L)]
        it1 = items_v[pl.ds(n * _SP + _L, _L)]
        v0 = it0 != PAD
        v1 = it1 != PAD
        p0 = jnp.where(v0, iota, -1)
        p1 = jnp.where(v1, iota + _L, -1)
        lastpos = _vred16(jnp.maximum(p0, p1), jnp.maximum)
        ones = jnp.ones((_L,), jnp.float32)
        plsc.addupdate_scatter(counts_v, [cbase + it0], ones, mask=v0)
        plsc.addupdate_scatter(counts_v, [cbase + it1], ones, mask=v1)
        li = plsc.load_gather(
            items_v, [n * _SP + jnp.maximum(lastpos, 0)]
        )
        plsc.addupdate_scatter(
            counts_v,
            [cbase + li],
            jnp.full((_L,), -1.0),
            mask=(iota == 0) & (lastpos >= 0),
        )
        return c

    lax.fori_loop(0, NB, body_a, 0)

    def do_chunk(ci, buf, sem):
        @pl.when(ci >= 2)
        def _():
            pltpu.make_async_copy(
                out_v.at[pl.ds(buf * CHW, CHW)],
                out_hbm.at[pl.ds(0, CHW)],
                sem,
            ).wait()

        for dn in range(CH):
            n_l = ci * CH + dn
            cbase = n_l * _CP
            rows = [
                counts_v[pl.ds(cbase + j * _L, _L)]
                for j in range(_CP // _L)
            ]
            vm = rows[0]
            for r in rows[1:]:
                vm = jnp.maximum(vm, r)
            m_v = _vred16(vm, jnp.maximum)
            se = jnp.zeros((_L,), jnp.float32)
            for r in rows:
                se = se + jnp.exp(r - m_v)
            sall = _vred16(se, jnp.add)

            def row_body(s, c):
                bans16 = bans_v[pl.ds(n_l * _SK + s * _KP, _KP)]
                bval = bans16 >= 0
                bidx = jnp.maximum(bans16, 0)
                ones = jnp.ones((_L,), jnp.float32)
                plsc.addupdate_scatter(cnt_v, [bidx], ones, mask=bval)
                mult = plsc.load_gather(cnt_v, [bidx])
                g = plsc.load_gather(counts_v, [cbase + bidx])
                contrib = jnp.where(bval, jnp.exp(g - m_v) / mult, 0.0)
                plsc.addupdate_scatter(cnt_v, [bidx], -ones, mask=bval)
                sban = _vred16(contrib, jnp.add)
                sunb = jnp.maximum(sall - sban, 1e-30)
                lse_v = m_v + _log16(sunb)
                obase = buf * CHW + (dn * 20 + s) * NUM_ITEMS
                for off in (0, 16, 32, 48, 64, 80, 96, 112, 128, 144, 160,
                            176, 184):
                    out_v[pl.ds(obase + off, _L)] = (
                        counts_v[pl.ds(cbase + off, _L)] - lse_v
                    )
                plsc.store_scatter(
                    out_v,
                    [obase + bidx],
                    jnp.full((_L,), NEG),
                    mask=bval,
                )
                return c

            lax.fori_loop(0, 20, row_body, 0)

        pltpu.async_copy(
            out_v.at[pl.ds(buf * CHW, CHW)],
            out_hbm.at[pl.ds((base_n + ci * CH) * 20 * NUM_ITEMS, CHW)],
            sem,
        )

    def pair_body(i, c):
        do_chunk(2 * i, 0, sem0)
        do_chunk(2 * i + 1, 1, sem1)
        return c

    lax.fori_loop(0, (NB // CH) // 2, pair_body, 0)
    pltpu.make_async_copy(
        out_v.at[pl.ds(0, CHW)], out_hbm.at[pl.ds(0, CHW)], sem0
    ).wait()
    pltpu.make_async_copy(
        out_v.at[pl.ds(CHW, CHW)], out_hbm.at[pl.ds(0, CHW)], sem1
    ).wait()


@jax.jit
def _spop_sc(ban_ids, item_ids):
    N, S = item_ids.shape
    K = ban_ids.shape[2]
    item_pad = jnp.pad(item_ids, ((0, 0), (0, _SP - S))).reshape(N * _SP)
    ban_pad = jnp.pad(
        ban_ids, ((0, 0), (0, 0), (0, _KP - K)), constant_values=-1
    ).reshape(N * _SK)
    CH = 4
    NB = N // _NW
    run = pl.kernel(
        _sc_body,
        out_type=jax.ShapeDtypeStruct((N * S * NUM_ITEMS,), jnp.float32),
        scratch_types=[
            pltpu.VMEM((NB * _SP,), jnp.int32),
            pltpu.VMEM((NB * _SK,), jnp.int32),
            pltpu.VMEM((NB * _CP,), jnp.float32),
            pltpu.VMEM((2 * CH * 20 * NUM_ITEMS,), jnp.float32),
            pltpu.VMEM((_CP,), jnp.float32),
            pltpu.SemaphoreType.DMA,
            pltpu.SemaphoreType.DMA,
        ],
        compiler_params=pltpu.CompilerParams(needs_layout_passes=False),
        mesh=plsc.VectorSubcoreMesh(core_axis_name="c", subcore_axis_name="s"),
    )
    pi = run(item_pad, ban_pad).reshape(N, S, NUM_ITEMS)
    return pi


def kernel(ban_ids, item_ids, aux1, aux2, aux3):
    pi = _spop_sc(ban_ids, item_ids)
    n, s = item_ids.shape
    v = jnp.zeros((n, s, 1), jnp.float32)
    return (pi, v)


# speedup vs baseline: 1.4300x; 1.0233x over previous
import functools

import jax
import jax.numpy as jnp
from jax.experimental import pallas as pl
from jax.experimental.pallas import tpu as pltpu

NUM_ITEMS = 200
PAD = 0
NEG = -1000000000.0


def _spop_block(item_ref, ban_ref, out_ref):
    B, S = item_ref.shape
    K = ban_ref.shape[2]
    C = NUM_ITEMS

    iota_i = jax.lax.broadcasted_iota(jnp.int32, (B, C), 1)
    counts = jnp.zeros((B, C), jnp.float32)
    last = jnp.full((B, 1), -1, jnp.int32)
    for s in range(S):
        col = item_ref[:, s].reshape(B, 1)
        valid = col != PAD
        counts = counts + jnp.where((iota_i == col) & valid, 1.0, 0.0)
        last = jnp.where(valid, col, last)
    counts = counts - jnp.where(iota_i == last, 1.0, 0.0)

    m = jnp.max(counts, axis=1, keepdims=True)
    exp_row = jnp.exp(counts - m)

    iota_f = jax.lax.broadcasted_iota(jnp.int32, (B, S, C), 2).astype(
        jnp.float32
    )
    d = ban_ref[:, :, 0].astype(jnp.float32)[:, :, None] - iota_f
    for k in range(1, K):
        d = d * (ban_ref[:, :, k].astype(jnp.float32)[:, :, None] - iota_f)
    banned = d == 0.0

    sum_unb = jnp.sum(
        jnp.where(banned, 0.0, exp_row[:, None, :]), axis=2
    )
    lse = m + jnp.log(sum_unb)
    out_ref[...] = (
        jnp.where(banned, NEG, counts[:, None, :]) - lse[:, :, None]
    )


@functools.partial(jax.jit, static_argnames=("interpret",))
def _spop(ban_ids, item_ids, interpret=False):
    N, S = item_ids.shape
    K = ban_ids.shape[2]
    B = 128
    grid = (N // B,)
    pi = pl.pallas_call(
        _spop_block,
        grid=grid,
        in_specs=[
            pl.BlockSpec((B, S), lambda i: (i, 0)),
            pl.BlockSpec((B, S, K), lambda i: (i, 0, 0)),
        ],
        out_specs=pl.BlockSpec((B, S, NUM_ITEMS), lambda i: (i, 0, 0)),
        out_shape=jax.ShapeDtypeStruct((N, S, NUM_ITEMS), jnp.float32),
        compiler_params=pltpu.CompilerParams(
            dimension_semantics=("parallel",),
        ),
        interpret=interpret,
    )(item_ids, ban_ids)
    return pi


from jax import lax
from jax.experimental.pallas import tpu_sc as plsc

_NC, _NS, _L = 2, 16, 16
_NW = _NC * _NS
_SP = 32
_KP = 16
_SK = 20 * _KP
_CP = 208
_LN2 = 0.6931471805599453


def _log16(x):
    bits = lax.bitcast_convert_type(x, jnp.int32)
    e = lax.shift_right_logical(bits, 23) - 127
    f = lax.bitcast_convert_type(
        jnp.bitwise_or(jnp.bitwise_and(bits, 0x007FFFFF), 0x3F800000),
        jnp.float32,
    )
    big = f > 1.4142135
    f = jnp.where(big, f * 0.5, f)
    e = e + jnp.where(big, 1, 0)
    z = (f - 1.0) / (f + 1.0)
    z2 = z * z
    p = 2.0 * z * (1.0 + z2 * (1.0 / 3.0 + z2 * (0.2 + z2 * (1.0 / 7.0))))
    return e.astype(jnp.float32) * _LN2 + p


def _vred16(x, op):
    iota = lax.iota(jnp.int32, _L)
    for sh in (8, 4, 2, 1):
        x = op(x, x.at[iota ^ sh].get(mode="promise_in_bounds"))
    return x


def _sc_body(item_hbm, ban_hbm, out_hbm, items_v, bans_v, counts_v, out_v,
             cnt_v, sem0, sem1):
    S, K = 20, 10
    N = item_hbm.shape[0] // S
    NB = N // _NW
    CH = 8
    CHW = CH * S * NUM_ITEMS
    UN = 4
    wid = lax.axis_index("s") * _NC + lax.axis_index("c")
    base_n = wid * NB
    iota = lax.iota(jnp.int32, _L)

    pltpu.sync_copy(item_hbm.at[pl.ds(base_n * S, NB * S)],
                    items_v.at[pl.ds(0, NB * S)])
    pltpu.sync_copy(ban_hbm.at[pl.ds(base_n * S * K, NB * S * K)],
                    bans_v.at[pl.ds(0, NB * S * K)])
    for u in range(UN):
        for j in range(_CP // _L):
            cnt_v[pl.ds(u * _CP + j * _L, _L)] = jnp.zeros((_L,), jnp.float32)

    def body_a(n, c):
        cbase = n * _CP
        for j in range(_CP // _L):
            counts_v[pl.ds(cbase + j * _L, _L)] = jnp.where(
                iota + (j * _L) < NUM_ITEMS, 0.0, -100000.0
            )
        it0 = plsc.load_gather(items_v, [n * S + iota])
        it1 = plsc.load_gather(items_v, [n * S + _L + iota])
        v0 = it0 != PAD
        v1 = (it1 != PAD) & (iota < S - _L)
        p0 = jnp.where(v0, iota, -1)
        p1 = jnp.where(v1, iota + _L, -1)
        lastpos = _vred16(jnp.maximum(p0, p1), jnp.maximum)
        ones = jnp.ones((_L,), jnp.float32)
        plsc.addupdate_scatter(counts_v, [cbase + it0], ones, mask=v0)
        plsc.addupdate_scatter(counts_v, [cbase + it1], ones, mask=v1)
        li = plsc.load_gather(items_v, [n * S + jnp.maximum(lastpos, 0)])
        plsc.addupdate_scatter(
            counts_v,
            [cbase + li],
            jnp.full((_L,), -1.0),
            mask=(iota == 0) & (lastpos >= 0),
        )
        return c

    lax.fori_loop(0, NB, body_a, 0)

    def do_chunk(ci, buf, sem):
        @pl.when(ci >= 2)
        def _():
            pltpu.make_async_copy(
                out_v.at[pl.ds(buf * CHW, CHW)],
                out_hbm.at[pl.ds(0, CHW)],
                sem,
            ).wait()

        for dn in range(CH):
            n_l = ci * CH + dn
            cbase = n_l * _CP
            rows = [
                counts_v[pl.ds(cbase + j * _L, _L)]
                for j in range(_CP // _L)
            ]
            vm = rows[0]
            for r in rows[1:]:
                vm = jnp.maximum(vm, r)
            m_v = _vred16(vm, jnp.maximum)
            se = jnp.zeros((_L,), jnp.float32)
            for r in rows:
                se = se + jnp.exp(r - m_v)
            sall = _vred16(se, jnp.add)
            ones = jnp.ones((_L,), jnp.float32)
            kmask = iota < K

            def row_one(s, u):
                bans16 = plsc.load_gather(bans_v, [n_l * S * K + s * K + iota])
                bidx = jnp.where(kmask, bans16, 0)
                plsc.addupdate_scatter(
                    cnt_v, [u * _CP + bidx], ones, mask=kmask
                )
                mult = plsc.load_gather(cnt_v, [u * _CP + bidx])
                g = plsc.load_gather(counts_v, [cbase + bidx])
                contrib = jnp.where(kmask, jnp.exp(g - m_v) / mult, 0.0)
                plsc.addupdate_scatter(
                    cnt_v, [u * _CP + bidx], -ones, mask=kmask
                )
                sban = _vred16(contrib, jnp.add)
                sunb = jnp.maximum(sall - sban, 1e-30)
                lse_v = m_v + _log16(sunb)
                obase = buf * CHW + (dn * S + s) * NUM_ITEMS
                for off in (0, 16, 32, 48, 64, 80, 96, 112, 128, 144, 160,
                            176, 184):
                    out_v[pl.ds(obase + off, _L)] = (
                        counts_v[pl.ds(cbase + off, _L)] - lse_v
                    )
                plsc.store_scatter(
                    out_v, [obase + bidx], jnp.full((_L,), NEG), mask=kmask
                )

            def row_group(g, c):
                for u in range(UN):
                    row_one(g * UN + u, u)
                return c

            lax.fori_loop(0, S // UN, row_group, 0)

        pltpu.async_copy(
            out_v.at[pl.ds(buf * CHW, CHW)],
            out_hbm.at[pl.ds((base_n + ci * CH) * S * NUM_ITEMS, CHW)],
            sem,
        )

    def pair_body(i, c):
        do_chunk(2 * i, 0, sem0)
        do_chunk(2 * i + 1, 1, sem1)
        return c

    lax.fori_loop(0, (NB // CH) // 2, pair_body, 0)
    pltpu.make_async_copy(
        out_v.at[pl.ds(0, CHW)], out_hbm.at[pl.ds(0, CHW)], sem0
    ).wait()
    pltpu.make_async_copy(
        out_v.at[pl.ds(CHW, CHW)], out_hbm.at[pl.ds(0, CHW)], sem1
    ).wait()


@jax.jit
def _spop_sc(ban_ids, item_ids):
    N, S = item_ids.shape
    K = ban_ids.shape[2]
    item_flat = item_ids.reshape(N * S)
    ban_flat = ban_ids.reshape(N * S * K)
    CH = 8
    NB = N // _NW
    run = pl.kernel(
        _sc_body,
        out_type=jax.ShapeDtypeStruct((N * S * NUM_ITEMS,), jnp.float32),
        scratch_types=[
            pltpu.VMEM((NB * S + _L,), jnp.int32),
            pltpu.VMEM((NB * S * K + _L,), jnp.int32),
            pltpu.VMEM((NB * _CP,), jnp.float32),
            pltpu.VMEM((2 * CH * S * NUM_ITEMS,), jnp.float32),
            pltpu.VMEM((4 * _CP,), jnp.float32),
            pltpu.SemaphoreType.DMA,
            pltpu.SemaphoreType.DMA,
        ],
        compiler_params=pltpu.CompilerParams(needs_layout_passes=False),
        mesh=plsc.VectorSubcoreMesh(core_axis_name="c", subcore_axis_name="s"),
    )
    pi = run(item_flat, ban_flat).reshape(N, S, NUM_ITEMS)
    return pi


def kernel(ban_ids, item_ids, aux1, aux2, aux3):
    pi = _spop_sc(ban_ids, item_ids)
    n, s = item_ids.shape
    v = jnp.zeros((n, s, 1), jnp.float32)
    return (pi, v)


# speedup vs baseline: 2.1417x; 1.4977x over previous
import functools

import jax
import jax.numpy as jnp
from jax.experimental import pallas as pl
from jax.experimental.pallas import tpu as pltpu

NUM_ITEMS = 200
PAD = 0
NEG = -1000000000.0


def _spop_block(item_ref, ban_ref, out_ref):
    B, S = item_ref.shape
    K = ban_ref.shape[2]
    C = NUM_ITEMS

    iota_i = jax.lax.broadcasted_iota(jnp.int32, (B, C), 1)
    counts = jnp.zeros((B, C), jnp.float32)
    last = jnp.full((B, 1), -1, jnp.int32)
    for s in range(S):
        col = item_ref[:, s].reshape(B, 1)
        valid = col != PAD
        counts = counts + jnp.where((iota_i == col) & valid, 1.0, 0.0)
        last = jnp.where(valid, col, last)
    counts = counts - jnp.where(iota_i == last, 1.0, 0.0)

    m = jnp.max(counts, axis=1, keepdims=True)
    exp_row = jnp.exp(counts - m)

    iota_f = jax.lax.broadcasted_iota(jnp.int32, (B, S, C), 2).astype(
        jnp.float32
    )
    d = ban_ref[:, :, 0].astype(jnp.float32)[:, :, None] - iota_f
    for k in range(1, K):
        d = d * (ban_ref[:, :, k].astype(jnp.float32)[:, :, None] - iota_f)
    banned = d == 0.0

    sum_unb = jnp.sum(
        jnp.where(banned, 0.0, exp_row[:, None, :]), axis=2
    )
    lse = m + jnp.log(sum_unb)
    out_ref[...] = (
        jnp.where(banned, NEG, counts[:, None, :]) - lse[:, :, None]
    )


@functools.partial(jax.jit, static_argnames=("interpret",))
def _spop(ban_ids, item_ids, interpret=False):
    N, S = item_ids.shape
    K = ban_ids.shape[2]
    B = 128
    grid = (N // B,)
    pi = pl.pallas_call(
        _spop_block,
        grid=grid,
        in_specs=[
            pl.BlockSpec((B, S), lambda i: (i, 0)),
            pl.BlockSpec((B, S, K), lambda i: (i, 0, 0)),
        ],
        out_specs=pl.BlockSpec((B, S, NUM_ITEMS), lambda i: (i, 0, 0)),
        out_shape=jax.ShapeDtypeStruct((N, S, NUM_ITEMS), jnp.float32),
        compiler_params=pltpu.CompilerParams(
            dimension_semantics=("parallel",),
        ),
        interpret=interpret,
    )(item_ids, ban_ids)
    return pi


from jax import lax
from jax.experimental.pallas import tpu_sc as plsc

_NC, _NS, _L = 2, 16, 16
_NW = _NC * _NS
_SP = 32
_KP = 16
_SK = 20 * _KP
_CP = 208
_LN2 = 0.6931471805599453


def _log16(x):
    bits = lax.bitcast_convert_type(x, jnp.int32)
    e = lax.shift_right_logical(bits, 23) - 127
    f = lax.bitcast_convert_type(
        jnp.bitwise_or(jnp.bitwise_and(bits, 0x007FFFFF), 0x3F800000),
        jnp.float32,
    )
    big = f > 1.4142135
    f = jnp.where(big, f * 0.5, f)
    e = e + jnp.where(big, 1, 0)
    z = (f - 1.0) / (f + 1.0)
    z2 = z * z
    p = 2.0 * z * (1.0 + z2 * (1.0 / 3.0 + z2 * (0.2 + z2 * (1.0 / 7.0))))
    return e.astype(jnp.float32) * _LN2 + p


def _vred16(x, op):
    iota = lax.iota(jnp.int32, _L)
    for sh in (8, 4, 2, 1):
        x = op(x, x.at[iota ^ sh].get(mode="promise_in_bounds"))
    return x


def _sc_body(item_hbm, ban_hbm, out_hbm, items_v, bans_v, counts_v, out_v,
             cnt_v, sem0, sem1):
    S, K = 20, 10
    N = item_hbm.shape[0] // S
    NB = N // _NW
    CH = 8
    CHW = CH * S * NUM_ITEMS
    UN = 4
    wid = lax.axis_index("s") * _NC + lax.axis_index("c")
    base_n = wid * NB
    iota = lax.iota(jnp.int32, _L)

    pltpu.sync_copy(item_hbm.at[pl.ds(base_n * S, NB * S)],
                    items_v.at[pl.ds(0, NB * S)])
    pltpu.sync_copy(ban_hbm.at[pl.ds(base_n * S * K, NB * S * K)],
                    bans_v.at[pl.ds(0, NB * S * K)])
    for u in range(UN):
        for j in range(_CP // _L):
            cnt_v[pl.ds(u * _CP + j * _L, _L)] = jnp.zeros((_L,), jnp.float32)

    def body_a(n, c):
        cbase = n * _CP
        for j in range(_CP // _L):
            counts_v[pl.ds(cbase + j * _L, _L)] = jnp.where(
                iota + (j * _L) < NUM_ITEMS, 0.0, -100000.0
            )
        it0 = plsc.load_gather(items_v, [n * S + iota])
        it1 = plsc.load_gather(items_v, [n * S + _L + iota])
        v0 = it0 != PAD
        v1 = (it1 != PAD) & (iota < S - _L)
        p0 = jnp.where(v0, iota, -1)
        p1 = jnp.where(v1, iota + _L, -1)
        lastpos = _vred16(jnp.maximum(p0, p1), jnp.maximum)
        ones = jnp.ones((_L,), jnp.float32)
        plsc.addupdate_scatter(counts_v, [cbase + it0], ones, mask=v0)
        plsc.addupdate_scatter(counts_v, [cbase + it1], ones, mask=v1)
        li = plsc.load_gather(items_v, [n * S + jnp.maximum(lastpos, 0)])
        plsc.addupdate_scatter(
            counts_v,
            [cbase + li],
            jnp.full((_L,), -1.0),
            mask=(iota == 0) & (lastpos >= 0),
        )
        return c

    lax.fori_loop(0, NB, body_a, 0)

    def do_chunk(ci, buf, sem):
        @pl.when(ci >= 2)
        def _():
            pltpu.make_async_copy(
                out_v.at[pl.ds(buf * CHW, CHW)],
                out_hbm.at[pl.ds(0, CHW)],
                sem,
            ).wait()

        for dn in range(CH):
            n_l = ci * CH + dn
            cbase = n_l * _CP
            rows = [
                counts_v[pl.ds(cbase + j * _L, _L)]
                for j in range(_CP // _L)
            ]
            vm = rows[0]
            for r in rows[1:]:
                vm = jnp.maximum(vm, r)
            m_v = _vred16(vm, jnp.maximum)
            se = jnp.zeros((_L,), jnp.float32)
            for r in rows:
                se = se + jnp.exp(r - m_v)
            sall = _vred16(se, jnp.add)
            ones = jnp.ones((_L,), jnp.float32)
            kmask = iota < K

            def row_one(s, u):
                bans16 = plsc.load_gather(bans_v, [n_l * S * K + s * K + iota])
                bidx = jnp.where(kmask, bans16, 0)
                plsc.addupdate_scatter(
                    cnt_v, [u * _CP + bidx], ones, mask=kmask
                )
                mult = plsc.load_gather(cnt_v, [u * _CP + bidx])
                g = plsc.load_gather(counts_v, [cbase + bidx])
                contrib = jnp.where(kmask, jnp.exp(g - m_v) / mult, 0.0)
                plsc.addupdate_scatter(
                    cnt_v, [u * _CP + bidx], -ones, mask=kmask
                )
                sban = _vred16(contrib, jnp.add)
                sunb = jnp.maximum(sall - sban, 1e-30)
                lse_v = m_v + _log16(sunb)
                obase = buf * CHW + (dn * S + s) * NUM_ITEMS
                for off in (0, 16, 32, 48, 64, 80, 96, 112, 128, 144, 160,
                            176, 184):
                    out_v[pl.ds(obase + off, _L)] = (
                        counts_v[pl.ds(cbase + off, _L)] - lse_v
                    )
                plsc.store_scatter(
                    out_v, [obase + bidx], jnp.full((_L,), NEG), mask=kmask
                )

            def row_group(g, c):
                for u in range(UN):
                    row_one(g * UN + u, u)
                return c

            lax.fori_loop(0, S // UN, row_group, 0)

        pltpu.async_copy(
            out_v.at[pl.ds(buf * CHW, CHW)],
            out_hbm.at[pl.ds((base_n + ci * CH) * S * NUM_ITEMS, CHW)],
            sem,
        )

    def pair_body(i, c):
        do_chunk(2 * i, 0, sem0)
        do_chunk(2 * i + 1, 1, sem1)
        return c

    lax.fori_loop(0, (NB // CH) // 2, pair_body, 0)
    pltpu.make_async_copy(
        out_v.at[pl.ds(0, CHW)], out_hbm.at[pl.ds(0, CHW)], sem0
    ).wait()
    pltpu.make_async_copy(
        out_v.at[pl.ds(CHW, CHW)], out_hbm.at[pl.ds(0, CHW)], sem1
    ).wait()


@jax.jit
def _spop_sc(ban_ids, item_ids):
    N, S = item_ids.shape
    K = ban_ids.shape[2]
    item_flat = item_ids.reshape(N * S)
    ban_flat = ban_ids.reshape(N * S * K)
    CH = 8
    NB = N // _NW
    run = pl.kernel(
        _sc_body,
        out_type=jax.ShapeDtypeStruct((N * S * NUM_ITEMS,), jnp.float32),
        scratch_types=[
            pltpu.VMEM((NB * S + _L,), jnp.int32),
            pltpu.VMEM((NB * S * K + _L,), jnp.int32),
            pltpu.VMEM((NB * _CP,), jnp.float32),
            pltpu.VMEM((2 * CH * S * NUM_ITEMS,), jnp.float32),
            pltpu.VMEM((4 * _CP,), jnp.float32),
            pltpu.SemaphoreType.DMA,
            pltpu.SemaphoreType.DMA,
        ],
        compiler_params=pltpu.CompilerParams(needs_layout_passes=False),
        mesh=plsc.VectorSubcoreMesh(core_axis_name="c", subcore_axis_name="s"),
    )
    pi = run(item_flat, ban_flat).reshape(N, S, NUM_ITEMS)
    return pi


_N_TC = 2560


@jax.jit
def _spop_hybrid(ban_ids, item_ids):
    pi_sc = _spop_sc(ban_ids[_N_TC:], item_ids[_N_TC:])
    pi_tc = _spop(ban_ids[:_N_TC], item_ids[:_N_TC])
    return jnp.concatenate([pi_tc, pi_sc], axis=0)


def kernel(ban_ids, item_ids, aux1, aux2, aux3):
    pi = _spop_hybrid(ban_ids, item_ids)
    n, s = item_ids.shape
    v = jnp.zeros((n, s, 1), jnp.float32)
    return (pi, v)
